# baseline (speedup 1.0000x reference)
import functools
import math

import jax
import jax.numpy as jnp
import numpy as np
from jax import lax
from jax.experimental import pallas as pl
from jax.experimental.pallas import tpu as pltpu
from jax.experimental.pallas import tpu_sc as plsc

VOCAB_SIZE = 32000
MODEL_DIM = 512
MAX_SEQ_LENGTH = 2048
SCALE = math.sqrt(MODEL_DIM)

NUM_CORES = 2
NUM_SUBCORES = 16
NUM_WORKERS = NUM_CORES * NUM_SUBCORES
LANES = 16

BATCH = 4
SEQ = 2048
POS_PER_WORKER = SEQ // NUM_WORKERS
CHUNK_POS = 16
NUM_CHUNKS = POS_PER_WORKER // CHUNK_POS
CHUNK_ROWS = CHUNK_POS * BATCH
VECS_PER_ROW = MODEL_DIM // LANES


def _pos_encoding_np(max_seq_length, model_dim):
    position = np.arange(max_seq_length)[:, None].astype(np.float32)
    div_term = np.exp(
        np.arange(0, model_dim, 2).astype(np.float32)
        * (-math.log(10000.0) / model_dim)
    )
    pe = np.zeros((max_seq_length, model_dim), dtype=np.float32)
    pe[:, 0::2] = np.sin(position * div_term)
    pe[:, 1::2] = np.cos(position * div_term)
    return pe


_PE = _pos_encoding_np(MAX_SEQ_LENGTH, MODEL_DIM)


def _sc_body(idx_hbm, table_hbm, pe_hbm, out_hbm, idx_v, rows_v, pe_v, sem):
    wid = lax.axis_index("s") * NUM_CORES + lax.axis_index("c")
    idx_base = wid * POS_PER_WORKER * BATCH
    pos_base = wid * POS_PER_WORKER

    pltpu.sync_copy(idx_hbm.at[pl.ds(idx_base, POS_PER_WORKER * BATCH)],
                    idx_v)

    for c in range(NUM_CHUNKS):
        idx_c = idx_v.at[pl.ds(c * CHUNK_ROWS, CHUNK_ROWS)]
        pltpu.async_copy(table_hbm.at[idx_c], rows_v, sem).wait()
        pltpu.sync_copy(
            pe_hbm.at[pl.ds(pos_base + c * CHUNK_POS, CHUNK_POS)], pe_v)

        def fma_body(p, _):
            for j in range(VECS_PER_ROW):
                pe_vec = pe_v[p, pl.ds(j * LANES, LANES)]
                for b in range(BATCH):
                    r = b * CHUNK_POS + p
                    sl = pl.ds(j * LANES, LANES)
                    rows_v[r, sl] = rows_v[r, sl] * SCALE + pe_vec
            return _

        lax.fori_loop(0, CHUNK_POS, fma_body, 0)

        for b in range(BATCH):
            out_row = b * SEQ + pos_base + c * CHUNK_POS
            pltpu.sync_copy(rows_v.at[pl.ds(b * CHUNK_POS, CHUNK_POS)],
                            out_hbm.at[pl.ds(out_row, CHUNK_POS)])


@functools.partial(jax.jit, static_argnames=())
def _embed(x, table):
    xr = x.reshape(BATCH, NUM_WORKERS, NUM_CHUNKS, CHUNK_POS)
    xr = xr.transpose(1, 2, 0, 3).reshape(BATCH * SEQ).astype(jnp.int32)
    pe = jnp.asarray(_PE)

    mesh = plsc.VectorSubcoreMesh(
        core_axis_name="c", subcore_axis_name="s",
        num_cores=NUM_CORES, num_subcores=NUM_SUBCORES)

    out = pl.kernel(
        _sc_body,
        out_type=jax.ShapeDtypeStruct((BATCH * SEQ, MODEL_DIM), jnp.float32),
        mesh=mesh,
        scratch_types=[
            pltpu.VMEM((POS_PER_WORKER * BATCH,), jnp.int32),
            pltpu.VMEM((CHUNK_ROWS, MODEL_DIM), jnp.float32),
            pltpu.VMEM((CHUNK_POS, MODEL_DIM), jnp.float32),
            pltpu.SemaphoreType.DMA,
        ],
    )(xr, table, pe)
    return out.reshape(BATCH, SEQ, MODEL_DIM)


def kernel(x, table):
    return _embed(x, table)


# speedup vs baseline: 1.1297x; 1.1297x over previous
import functools
import math

import jax
import jax.numpy as jnp
import numpy as np
from jax import lax
from jax.experimental import pallas as pl
from jax.experimental.pallas import tpu as pltpu
from jax.experimental.pallas import tpu_sc as plsc

VOCAB_SIZE = 32000
MODEL_DIM = 512
MAX_SEQ_LENGTH = 2048
SCALE = math.sqrt(MODEL_DIM)

NUM_CORES = 2
NUM_SUBCORES = 16
NUM_WORKERS = NUM_CORES * NUM_SUBCORES
LANES = 16

BATCH = 4
SEQ = 2048
POS_PER_WORKER = SEQ // NUM_WORKERS
CHUNK_POS = 16
NUM_CHUNKS = POS_PER_WORKER // CHUNK_POS
CHUNK_ROWS = CHUNK_POS * BATCH
VECS_PER_ROW = MODEL_DIM // LANES


def _pos_encoding_np(max_seq_length, model_dim):
    position = np.arange(max_seq_length)[:, None].astype(np.float32)
    div_term = np.exp(
        np.arange(0, model_dim, 2).astype(np.float32)
        * (-math.log(10000.0) / model_dim)
    )
    pe = np.zeros((max_seq_length, model_dim), dtype=np.float32)
    pe[:, 0::2] = np.sin(position * div_term)
    pe[:, 1::2] = np.cos(position * div_term)
    return pe


_PE = _pos_encoding_np(MAX_SEQ_LENGTH, MODEL_DIM)


def _sc_body(idx_hbm, table_hbm, pe_hbm, out_hbm, idx_v,
             rows0, rows1, pe0, pe1,
             sem_g0, sem_g1, sem_p0, sem_p1, sem_s0, sem_s1):
    wid = lax.axis_index("s") * NUM_CORES + lax.axis_index("c")
    idx_base = wid * POS_PER_WORKER * BATCH
    pos_base = wid * POS_PER_WORKER

    rows = [rows0, rows1]
    pe = [pe0, pe1]
    sem_g = [sem_g0, sem_g1]
    sem_p = [sem_p0, sem_p1]
    sem_s = [sem_s0, sem_s1]

    pltpu.sync_copy(idx_hbm.at[pl.ds(idx_base, POS_PER_WORKER * BATCH)],
                    idx_v)

    def fire_chunk(c):
        b = c % 2
        idx_c = idx_v.at[pl.ds(c * CHUNK_ROWS, CHUNK_ROWS)]
        g = pltpu.async_copy(table_hbm.at[idx_c], rows[b], sem_g[b])
        p = pltpu.async_copy(
            pe_hbm.at[pl.ds(pos_base + c * CHUNK_POS, CHUNK_POS)],
            pe[b], sem_p[b])
        return g, p

    pending = {0: fire_chunk(0)}
    stores = {}

    for c in range(NUM_CHUNKS):
        b = c % 2
        if c - 1 in stores:
            for s in stores.pop(c - 1):
                s.wait()
        if c + 1 < NUM_CHUNKS:
            pending[c + 1] = fire_chunk(c + 1)
        g, p = pending.pop(c)
        g.wait()
        p.wait()

        @plsc.parallel_loop(0, CHUNK_POS, unroll=2)
        def fma_body(pp):
            for j in range(VECS_PER_ROW):
                sl = pl.ds(j * LANES, LANES)
                pe_vec = pe[b][pp, sl]
                for bb in range(BATCH):
                    r = bb * CHUNK_POS + pp
                    rows[b][r, sl] = rows[b][r, sl] * SCALE + pe_vec

        st = []
        for bb in range(BATCH):
            out_row = bb * SEQ + pos_base + c * CHUNK_POS
            st.append(pltpu.async_copy(
                rows[b].at[pl.ds(bb * CHUNK_POS, CHUNK_POS)],
                out_hbm.at[pl.ds(out_row, CHUNK_POS)], sem_s[b]))
        stores[c] = st

    for c in sorted(stores):
        for s in stores[c]:
            s.wait()


@functools.partial(jax.jit, static_argnames=())
def _embed(x, table):
    xr = x.reshape(BATCH, NUM_WORKERS, NUM_CHUNKS, CHUNK_POS)
    xr = xr.transpose(1, 2, 0, 3).reshape(BATCH * SEQ).astype(jnp.int32)
    pe = jnp.asarray(_PE)

    mesh = plsc.VectorSubcoreMesh(
        core_axis_name="c", subcore_axis_name="s",
        num_cores=NUM_CORES, num_subcores=NUM_SUBCORES)

    out = pl.kernel(
        _sc_body,
        out_type=jax.ShapeDtypeStruct((BATCH * SEQ, MODEL_DIM), jnp.float32),
        mesh=mesh,
        scratch_types=[
            pltpu.VMEM((POS_PER_WORKER * BATCH,), jnp.int32),
            pltpu.VMEM((CHUNK_ROWS, MODEL_DIM), jnp.float32),
            pltpu.VMEM((CHUNK_ROWS, MODEL_DIM), jnp.float32),
            pltpu.VMEM((CHUNK_POS, MODEL_DIM), jnp.float32),
            pltpu.VMEM((CHUNK_POS, MODEL_DIM), jnp.float32),
            pltpu.SemaphoreType.DMA,
            pltpu.SemaphoreType.DMA,
            pltpu.SemaphoreType.DMA,
            pltpu.SemaphoreType.DMA,
            pltpu.SemaphoreType.DMA,
            pltpu.SemaphoreType.DMA,
        ],
    )(xr, table, pe)
    return out.reshape(BATCH, SEQ, MODEL_DIM)


def kernel(x, table):
    return _embed(x, table)


# speedup vs baseline: 1.2092x; 1.0704x over previous
import functools
import math

import jax
import jax.numpy as jnp
import numpy as np
from jax import lax
from jax.experimental import pallas as pl
from jax.experimental.pallas import tpu as pltpu
from jax.experimental.pallas import tpu_sc as plsc

VOCAB_SIZE = 32000
MODEL_DIM = 512
MAX_SEQ_LENGTH = 2048
SCALE = math.sqrt(MODEL_DIM)

NUM_CORES = 2
NUM_SUBCORES = 16
NUM_WORKERS = NUM_CORES * NUM_SUBCORES

BATCH = 4
SEQ = 2048
TOTAL_ROWS = BATCH * SEQ
ROWS_PER_WORKER = TOTAL_ROWS // NUM_WORKERS
CHUNK_ROWS = 64
NUM_CHUNKS = ROWS_PER_WORKER // CHUNK_ROWS

SEQ_BLOCK = 256
SEQ_BLOCKS = SEQ // SEQ_BLOCK


def _pos_encoding_np(max_seq_length, model_dim):
    position = np.arange(max_seq_length)[:, None].astype(np.float32)
    div_term = np.exp(
        np.arange(0, model_dim, 2).astype(np.float32)
        * (-math.log(10000.0) / model_dim)
    )
    pe = np.zeros((max_seq_length, model_dim), dtype=np.float32)
    pe[:, 0::2] = np.sin(position * div_term)
    pe[:, 1::2] = np.cos(position * div_term)
    return pe


_PE = _pos_encoding_np(MAX_SEQ_LENGTH, MODEL_DIM)


def _sc_gather_body(idx_hbm, table_hbm, out_hbm, idx_v, rows0, rows1,
                    sem_g0, sem_g1, sem_s0, sem_s1):
    wid = lax.axis_index("s") * NUM_CORES + lax.axis_index("c")
    base = wid * ROWS_PER_WORKER

    rows = [rows0, rows1]
    sem_g = [sem_g0, sem_g1]
    sem_s = [sem_s0, sem_s1]

    pltpu.sync_copy(idx_hbm.at[pl.ds(base, ROWS_PER_WORKER)], idx_v)

    def fire_gather(c):
        b = c % 2
        return pltpu.async_copy(
            table_hbm.at[idx_v.at[pl.ds(c * CHUNK_ROWS, CHUNK_ROWS)]],
            rows[b], sem_g[b])

    pending = {0: fire_gather(0)}
    stores = {}

    for c in range(NUM_CHUNKS):
        b = c % 2
        if c - 1 in stores:
            stores.pop(c - 1).wait()
        if c + 1 < NUM_CHUNKS:
            pending[c + 1] = fire_gather(c + 1)
        pending.pop(c).wait()
        stores[c] = pltpu.async_copy(
            rows[b], out_hbm.at[pl.ds(base + c * CHUNK_ROWS, CHUNK_ROWS)],
            sem_s[b])

    for c in sorted(stores):
        stores[c].wait()


def _fma_kernel(g_ref, pe_ref, o_ref):
    o_ref[...] = g_ref[...] * SCALE + pe_ref[...]


@jax.jit
def _embed(x, table):
    x_flat = x.reshape(TOTAL_ROWS).astype(jnp.int32)
    pe = jnp.asarray(_PE)

    mesh = plsc.VectorSubcoreMesh(
        core_axis_name="c", subcore_axis_name="s",
        num_cores=NUM_CORES, num_subcores=NUM_SUBCORES)

    gathered = pl.kernel(
        _sc_gather_body,
        out_type=jax.ShapeDtypeStruct((TOTAL_ROWS, MODEL_DIM), jnp.float32),
        mesh=mesh,
        scratch_types=[
            pltpu.VMEM((ROWS_PER_WORKER,), jnp.int32),
            pltpu.VMEM((CHUNK_ROWS, MODEL_DIM), jnp.float32),
            pltpu.VMEM((CHUNK_ROWS, MODEL_DIM), jnp.float32),
            pltpu.SemaphoreType.DMA,
            pltpu.SemaphoreType.DMA,
            pltpu.SemaphoreType.DMA,
            pltpu.SemaphoreType.DMA,
        ],
    )(x_flat, table)

    g3 = gathered.reshape(BATCH, SEQ, MODEL_DIM)
    out = pl.pallas_call(
        _fma_kernel,
        out_shape=jax.ShapeDtypeStruct((BATCH, SEQ, MODEL_DIM), jnp.float32),
        grid=(SEQ_BLOCKS, BATCH),
        in_specs=[
            pl.BlockSpec((1, SEQ_BLOCK, MODEL_DIM), lambda s, b: (b, s, 0)),
            pl.BlockSpec((SEQ_BLOCK, MODEL_DIM), lambda s, b: (s, 0)),
        ],
        out_specs=pl.BlockSpec((1, SEQ_BLOCK, MODEL_DIM), lambda s, b: (b, s, 0)),
        compiler_params=pltpu.CompilerParams(
            dimension_semantics=("arbitrary", "arbitrary"),
        ),
    )(g3, pe)
    return out


def kernel(x, table):
    return _embed(x, table)


# speedup vs baseline: 1.4126x; 1.1682x over previous
import functools
import math

import jax
import jax.numpy as jnp
import numpy as np
from jax import lax
from jax.experimental import pallas as pl
from jax.experimental.pallas import tpu as pltpu
from jax.experimental.pallas import tpu_sc as plsc

VOCAB_SIZE = 32000
MODEL_DIM = 512
MAX_SEQ_LENGTH = 2048
SCALE = math.sqrt(MODEL_DIM)

NUM_CORES = 2
NUM_SUBCORES = 16
NUM_WORKERS = NUM_CORES * NUM_SUBCORES

BATCH = 4
SEQ = 2048
TOTAL_ROWS = BATCH * SEQ
ROWS_PER_WORKER = TOTAL_ROWS // NUM_WORKERS
CHUNK_ROWS = 64
NUM_CHUNKS = ROWS_PER_WORKER // CHUNK_ROWS

ROW_BLOCK = 1024
ROW_BLOCKS = TOTAL_ROWS // ROW_BLOCK
PE_BLOCKS = SEQ // ROW_BLOCK


def _pos_encoding_np(max_seq_length, model_dim):
    position = np.arange(max_seq_length)[:, None].astype(np.float32)
    div_term = np.exp(
        np.arange(0, model_dim, 2).astype(np.float32)
        * (-math.log(10000.0) / model_dim)
    )
    pe = np.zeros((max_seq_length, model_dim), dtype=np.float32)
    pe[:, 0::2] = np.sin(position * div_term)
    pe[:, 1::2] = np.cos(position * div_term)
    return pe


_PE = _pos_encoding_np(MAX_SEQ_LENGTH, MODEL_DIM)


def _sc_gather_body(idx_hbm, table_hbm, out_hbm, idx_v, rows0, rows1,
                    sem_g0, sem_g1, sem_s0, sem_s1):
    wid = lax.axis_index("s") * NUM_CORES + lax.axis_index("c")
    base = wid * ROWS_PER_WORKER

    rows = [rows0, rows1]
    sem_g = [sem_g0, sem_g1]
    sem_s = [sem_s0, sem_s1]

    pltpu.sync_copy(idx_hbm.at[pl.ds(base, ROWS_PER_WORKER)], idx_v)

    def fire_gather(c):
        b = c % 2
        return pltpu.async_copy(
            table_hbm.at[idx_v.at[pl.ds(c * CHUNK_ROWS, CHUNK_ROWS)]],
            rows[b], sem_g[b])

    pending = {0: fire_gather(0)}
    stores = {}

    for c in range(NUM_CHUNKS):
        b = c % 2
        if c - 1 in stores:
            stores.pop(c - 1).wait()
        if c + 1 < NUM_CHUNKS:
            pending[c + 1] = fire_gather(c + 1)
        pending.pop(c).wait()
        stores[c] = pltpu.async_copy(
            rows[b], out_hbm.at[pl.ds(base + c * CHUNK_ROWS, CHUNK_ROWS)],
            sem_s[b])

    for c in sorted(stores):
        stores[c].wait()


def _fma_kernel(g_ref, pe_ref, o_ref):
    o_ref[...] = g_ref[...] * SCALE + pe_ref[...]


@jax.jit
def _embed(x, table):
    x_flat = x.reshape(TOTAL_ROWS).astype(jnp.int32)
    pe = jnp.asarray(_PE)

    mesh = plsc.VectorSubcoreMesh(
        core_axis_name="c", subcore_axis_name="s",
        num_cores=NUM_CORES, num_subcores=NUM_SUBCORES)

    gathered = pl.kernel(
        _sc_gather_body,
        out_type=jax.ShapeDtypeStruct((TOTAL_ROWS, MODEL_DIM), jnp.float32),
        mesh=mesh,
        scratch_types=[
            pltpu.VMEM((ROWS_PER_WORKER,), jnp.int32),
            pltpu.VMEM((CHUNK_ROWS, MODEL_DIM), jnp.float32),
            pltpu.VMEM((CHUNK_ROWS, MODEL_DIM), jnp.float32),
            pltpu.SemaphoreType.DMA,
            pltpu.SemaphoreType.DMA,
            pltpu.SemaphoreType.DMA,
            pltpu.SemaphoreType.DMA,
        ],
    )(x_flat, table)

    out = pl.pallas_call(
        _fma_kernel,
        out_shape=jax.ShapeDtypeStruct((TOTAL_ROWS, MODEL_DIM), jnp.float32),
        grid=(ROW_BLOCKS,),
        in_specs=[
            pl.BlockSpec((ROW_BLOCK, MODEL_DIM), lambda i: (i, 0)),
            pl.BlockSpec((ROW_BLOCK, MODEL_DIM), lambda i: (i % PE_BLOCKS, 0)),
        ],
        out_specs=pl.BlockSpec((ROW_BLOCK, MODEL_DIM), lambda i: (i, 0)),
        compiler_params=pltpu.CompilerParams(
            dimension_semantics=("arbitrary",),
        ),
    )(gathered, pe)
    return out.reshape(BATCH, SEQ, MODEL_DIM)


def kernel(x, table):
    return _embed(x, table)
